# baseline (speedup 1.0000x reference)
import functools

import jax
import jax.numpy as jnp
from jax import lax
from jax.experimental import pallas as pl
from jax.experimental.pallas import tpu as pltpu
from jax.experimental.pallas import tpu_sc as plsc

BATCH = 16384
D = 32
NC = 2
NS = 16
NW = NC * NS
BPW = BATCH // NW
CH = 128
NCH = BPW // CH


def _gather_tiled_body(ids, tab, out, idx_v, rows_a, rows_b, sem_a, sem_b):
    wid = lax.axis_index("s") * NC + lax.axis_index("c")
    base = wid * BPW
    pltpu.sync_copy(ids.at[pl.ds(base, BPW)], idx_v)

    bufs = (rows_a, rows_b)
    sems = (sem_a, sem_b)

    def issue_chunk(ch, slot):
        buf = bufs[slot]
        sem = sems[slot]

        def body(g, _):
            vec = idx_v[pl.ds(ch * CH + g * 16, 16)]
            off = g * 16
            for l in range(16):
                idx = vec[l]
                pltpu.make_async_copy(tab.at[pl.ds(idx, 1)],
                                      buf.at[pl.ds(off + l, 1)], sem).start()
            return _

        lax.fori_loop(0, CH // 16, body, None)

    def drain_chunk(ch, slot):
        pltpu.make_async_copy(tab.at[pl.ds(0, CH)], bufs[slot],
                              sems[slot]).wait()
        pltpu.sync_copy(bufs[slot], out.at[pl.ds(base + ch * CH, CH)])

    for step in range(NCH + 2):
        slot = step % 2
        if step >= 2:
            drain_chunk(step - 2, slot)
        if step < NCH:
            issue_chunk(step, slot)


_sc_gather_tiled = pl.kernel(
    _gather_tiled_body,
    out_type=jax.ShapeDtypeStruct((BATCH, D), jnp.float32),
    mesh=plsc.VectorSubcoreMesh(core_axis_name="c", subcore_axis_name="s"),
    scratch_types=[
        pltpu.VMEM((BPW,), jnp.int32),
        pltpu.VMEM((CH, D), jnp.float32),
        pltpu.VMEM((CH, D), jnp.float32),
        pltpu.SemaphoreType.DMA,
        pltpu.SemaphoreType.DMA,
    ],
    compiler_params=pltpu.CompilerParams(use_tc_tiling_on_sc=True),
)


def _gather_dense_body(ids, tab, out, idx_v, rows_v, sem):
    wid = lax.axis_index("s") * NC + lax.axis_index("c")
    base = wid * BPW
    pltpu.sync_copy(ids.at[pl.ds(base, BPW)], idx_v)
    pltpu.async_copy(tab.at[idx_v], rows_v, sem).wait()
    pltpu.sync_copy(rows_v, out.at[pl.ds(base, BPW)])


_sc_gather_dense = pl.kernel(
    _gather_dense_body,
    out_type=jax.ShapeDtypeStruct((BATCH, D), jnp.float32),
    mesh=plsc.VectorSubcoreMesh(core_axis_name="c", subcore_axis_name="s"),
    scratch_types=[
        pltpu.VMEM((BPW,), jnp.int32),
        pltpu.VMEM((BPW, D), jnp.float32),
        pltpu.SemaphoreType.DMA,
    ],
    compiler_params=pltpu.CompilerParams(use_tc_tiling_on_sc=False),
)


def _mlp_body(u_ref, i_ref, w1u_ref, w1i_ref, b1_ref, w2_ref, b2_ref,
              w3_ref, b3_ref, out_ref):
    u = u_ref[...]
    i = i_ref[...]
    h = jnp.dot(u, w1u_ref[...], preferred_element_type=jnp.float32)
    h = h + jnp.dot(i, w1i_ref[...], preferred_element_type=jnp.float32)
    h = jnp.maximum(h + b1_ref[...], 0.0)
    h2 = jnp.dot(h, w2_ref[...], preferred_element_type=jnp.float32)
    h2 = jnp.maximum(h2 + b2_ref[...], 0.0)
    out_ref[...] = jnp.sum(h2 * w3_ref[...], axis=1, keepdims=True) + b3_ref[...]


_mlp = pl.pallas_call(
    _mlp_body,
    out_shape=jax.ShapeDtypeStruct((BATCH, 1), jnp.float32),
)


def kernel(U_ids, I_ids, user_table, item_table, W1, b1, W2, b2, W3, b3):
    u_ids = U_ids.astype(jnp.int32)
    i_ids = I_ids.astype(jnp.int32)
    i_emb = _sc_gather_dense(i_ids, item_table)
    u_emb = _sc_gather_tiled(u_ids, user_table)
    return _mlp(u_emb, i_emb, W1[:D], W1[D:], b1.reshape(1, D),
                W2, b2.reshape(1, 8), W3.reshape(1, 8), b3.reshape(1, 1))
